# baseline (speedup 1.0000x reference)
import jax
import jax.numpy as jnp
from jax.experimental import pallas as pl


def kernel(x):
    xt = jnp.moveaxis(x, 1, -1)
    _, idx = jax.lax.top_k(xt, 64)
    idx = jnp.sort(idx, axis=-1)
    out = jnp.take_along_axis(xt, idx, axis=-1)
    return jnp.moveaxis(out, -1, 1)


# speedup vs baseline: 3.9177x; 3.9177x over previous
import functools

import jax
import jax.numpy as jnp
from jax import lax
from jax.experimental import pallas as pl
from jax.experimental.pallas import tpu as pltpu
from jax.experimental.pallas import tpu_sc as plsc

_K = 64
_B = 4
_S = 4096
_C = 1024
_MININT = -(2**31)

_NC = 2
_NS = 16
_NW = _NC * _NS
_CPW = _C // (_NW // _B)
_GRP = _CPW // 16


def _tc_threshold_body(x_ref, thr_ref, neq_ref):
    x = x_ref[0]
    b = lax.bitcast_convert_type(x, jnp.int32)
    s = jnp.where(b < 0, jnp.bitwise_xor(jnp.bitwise_not(b), _MININT), b)

    def bit_step(i, prefix):
        bit = jnp.left_shift(jnp.int32(1), 31 - i)
        cand = jnp.bitwise_or(prefix, bit)
        cand_s = jnp.bitwise_xor(cand, _MININT)
        cnt = jnp.sum((s >= cand_s).astype(jnp.int32), axis=0, keepdims=True)
        return jnp.where(cnt >= _K, cand, prefix)

    prefix = lax.fori_loop(0, 32, bit_step, jnp.zeros((1, 128), jnp.int32))
    t_s = jnp.bitwise_xor(prefix, _MININT)
    n_gt = jnp.sum((s > t_s).astype(jnp.int32), axis=0, keepdims=True)
    b_thr = jnp.where(t_s < 0, jnp.bitwise_not(jnp.bitwise_xor(t_s, _MININT)), t_s)
    thr_ref[...] = lax.bitcast_convert_type(b_thr, jnp.float32)[None, None]
    neq_ref[...] = (_K - n_gt)[None, None]


_CHUNK = 512


def _sc_select_body(x_hbm, thr_hbm, neq_hbm, out_hbm, xbuf, obuf, tbuf, nbuf):
    wid = lax.axis_index("c") * _NS + lax.axis_index("s")
    b = wid // (_NW // _B)
    cbase = (wid % (_NW // _B)) * _CPW
    coff = b * _C + cbase
    lane = lax.iota(jnp.int32, 16)

    pltpu.sync_copy(thr_hbm.at[pl.ds(coff, _CPW)], tbuf)
    pltpu.sync_copy(neq_hbm.at[pl.ds(coff, _CPW)], nbuf)

    zero = jnp.zeros((16,), jnp.int32)
    st0 = (zero,) * (2 * _GRP)

    def chunk(ch, st):
        pltpu.sync_copy(
            x_hbm.at[b, pl.ds(ch * _CHUNK, _CHUNK), pl.ds(cbase, _CPW)], xbuf)
        new_st = []
        for g in range(_GRP):
            thr = tbuf[pl.ds(g * 16, 16)]
            neq = nbuf[pl.ds(g * 16, 16)]
            colbase = (g * 16 + lane) * _K

            def step(i, s2, thr=thr, neq=neq, colbase=colbase, g=g):
                cnt_eq, cnt_out = s2
                v = xbuf[i, pl.ds(g * 16, 16)]
                gt = v > thr
                eq = v == thr
                sel = jnp.logical_or(gt, jnp.logical_and(eq, cnt_eq < neq))
                plsc.store_scatter(obuf, [colbase + cnt_out], v, mask=sel)
                return (cnt_eq + eq.astype(jnp.int32),
                        cnt_out + sel.astype(jnp.int32))

            new_st += list(lax.fori_loop(
                0, _CHUNK, step, (st[2 * g], st[2 * g + 1])))
        return tuple(new_st)

    lax.fori_loop(0, _S // _CHUNK, chunk, st0)
    pltpu.sync_copy(obuf, out_hbm.at[pl.ds(wid * (_CPW * _K), _CPW * _K)])


@functools.cache
def _sc_select():
    return pl.kernel(
        _sc_select_body,
        out_type=jax.ShapeDtypeStruct((_NW * _CPW * _K,), jnp.float32),
        mesh=plsc.VectorSubcoreMesh(
            core_axis_name="c", subcore_axis_name="s",
            num_cores=_NC, num_subcores=_NS,
        ),
        compiler_params=pltpu.CompilerParams(needs_layout_passes=False),
        scratch_types=[
            pltpu.VMEM((_CHUNK, _CPW), jnp.float32),
            pltpu.VMEM((_CPW * _K,), jnp.float32),
            pltpu.VMEM((_CPW,), jnp.float32),
            pltpu.VMEM((_CPW,), jnp.int32),
        ],
    )


def kernel(x):
    thr4, neq4 = pl.pallas_call(
        _tc_threshold_body,
        grid=(_B, _C // 128),
        in_specs=[pl.BlockSpec((1, _S, 128), lambda b, cb: (b, 0, cb))],
        out_specs=[
            pl.BlockSpec((1, 1, 1, 128), lambda b, cb: (b, cb, 0, 0)),
            pl.BlockSpec((1, 1, 1, 128), lambda b, cb: (b, cb, 0, 0)),
        ],
        out_shape=[
            jax.ShapeDtypeStruct((_B, _C // 128, 1, 128), jnp.float32),
            jax.ShapeDtypeStruct((_B, _C // 128, 1, 128), jnp.int32),
        ],
    )(x)
    thr = thr4.reshape(_B * _C)
    neq = neq4.reshape(_B * _C)
    out_flat = _sc_select()(x, thr, neq)
    out = out_flat.reshape(_B, _NW // _B, _CPW, _K)
    return out.transpose(0, 3, 1, 2).reshape(_B, _K, _C)


# speedup vs baseline: 4.0853x; 1.0428x over previous
import functools

import jax
import jax.numpy as jnp
from jax import lax
from jax.experimental import pallas as pl
from jax.experimental.pallas import tpu as pltpu
from jax.experimental.pallas import tpu_sc as plsc

_K = 64
_B = 4
_S = 4096
_C = 1024
_MININT = -(2**31)

_NC = 2
_NS = 16
_NW = _NC * _NS
_CPW = _C // (_NW // _B)
_GRP = _CPW // 16


def _tc_threshold_body(x_ref, thr_ref, neq_ref):
    x = x_ref[0]
    b = lax.bitcast_convert_type(x, jnp.int32)
    s = jnp.where(b < 0, jnp.bitwise_xor(jnp.bitwise_not(b), _MININT), b)

    s4 = s.reshape(64, 64, 128)

    def _count_ge(c):
        m = (s4 >= c.reshape(1, 1, 128)).astype(jnp.int32)
        return jnp.sum(jnp.sum(m, axis=0), axis=0, keepdims=True)

    def bit_step(i, prefix):
        bit = jnp.left_shift(jnp.int32(1), 31 - i)
        cand = jnp.bitwise_or(prefix, bit)
        cand_s = jnp.bitwise_xor(cand, _MININT)
        cnt = _count_ge(cand_s)
        return jnp.where(cnt >= _K, cand, prefix)

    prefix = lax.fori_loop(0, 32, bit_step, jnp.zeros((1, 128), jnp.int32))
    t_s = jnp.bitwise_xor(prefix, _MININT)
    m_gt = (s4 > t_s.reshape(1, 1, 128)).astype(jnp.int32)
    n_gt = jnp.sum(jnp.sum(m_gt, axis=0), axis=0, keepdims=True)
    b_thr = jnp.where(t_s < 0, jnp.bitwise_not(jnp.bitwise_xor(t_s, _MININT)), t_s)
    thr_ref[...] = lax.bitcast_convert_type(b_thr, jnp.float32)[None, None]
    neq_ref[...] = (_K - n_gt)[None, None]


_CHUNK = 512


def _sc_select_body(x_hbm, thr_hbm, neq_hbm, out_hbm, xbuf, obuf, tbuf, nbuf):
    wid = lax.axis_index("c") * _NS + lax.axis_index("s")
    b = wid // (_NW // _B)
    cbase = (wid % (_NW // _B)) * _CPW
    coff = b * _C + cbase
    lane = lax.iota(jnp.int32, 16)

    pltpu.sync_copy(thr_hbm.at[pl.ds(coff, _CPW)], tbuf)
    pltpu.sync_copy(neq_hbm.at[pl.ds(coff, _CPW)], nbuf)

    zero = jnp.zeros((16,), jnp.int32)
    st0 = (zero,) * (2 * _GRP)

    thrs = [tbuf[pl.ds(g * 16, 16)] for g in range(_GRP)]
    neqs = [nbuf[pl.ds(g * 16, 16)] for g in range(_GRP)]
    colbases = [(g * 16 + lane) * _K for g in range(_GRP)]

    def chunk(ch, st):
        pltpu.sync_copy(
            x_hbm.at[b, pl.ds(ch * _CHUNK, _CHUNK), pl.ds(cbase, _CPW)], xbuf)

        def step(i, s2):
            new = []
            for g in range(_GRP):
                cnt_eq, cnt_out = s2[2 * g], s2[2 * g + 1]
                v = xbuf[i, pl.ds(g * 16, 16)]
                gt = v > thrs[g]
                eq = v == thrs[g]
                sel = jnp.logical_or(gt, jnp.logical_and(eq, cnt_eq < neqs[g]))
                plsc.store_scatter(obuf, [colbases[g] + cnt_out], v, mask=sel)
                new += [cnt_eq + eq.astype(jnp.int32),
                        cnt_out + sel.astype(jnp.int32)]
            return tuple(new)

        return lax.fori_loop(0, _CHUNK, step, st)

    lax.fori_loop(0, _S // _CHUNK, chunk, st0)
    pltpu.sync_copy(obuf, out_hbm.at[pl.ds(wid * (_CPW * _K), _CPW * _K)])


@functools.cache
def _sc_select():
    return pl.kernel(
        _sc_select_body,
        out_type=jax.ShapeDtypeStruct((_NW * _CPW * _K,), jnp.float32),
        mesh=plsc.VectorSubcoreMesh(
            core_axis_name="c", subcore_axis_name="s",
            num_cores=_NC, num_subcores=_NS,
        ),
        compiler_params=pltpu.CompilerParams(needs_layout_passes=False),
        scratch_types=[
            pltpu.VMEM((_CHUNK, _CPW), jnp.float32),
            pltpu.VMEM((_CPW * _K,), jnp.float32),
            pltpu.VMEM((_CPW,), jnp.float32),
            pltpu.VMEM((_CPW,), jnp.int32),
        ],
    )


def kernel(x):
    thr4, neq4 = pl.pallas_call(
        _tc_threshold_body,
        grid=(_B, _C // 128),
        in_specs=[pl.BlockSpec((1, _S, 128), lambda b, cb: (b, 0, cb))],
        out_specs=[
            pl.BlockSpec((1, 1, 1, 128), lambda b, cb: (b, cb, 0, 0)),
            pl.BlockSpec((1, 1, 1, 128), lambda b, cb: (b, cb, 0, 0)),
        ],
        out_shape=[
            jax.ShapeDtypeStruct((_B, _C // 128, 1, 128), jnp.float32),
            jax.ShapeDtypeStruct((_B, _C // 128, 1, 128), jnp.int32),
        ],
    )(x)
    thr = thr4.reshape(_B * _C)
    neq = neq4.reshape(_B * _C)
    out_flat = _sc_select()(x, thr, neq)
    out = out_flat.reshape(_B, _NW // _B, _CPW, _K)
    return out.transpose(0, 3, 1, 2).reshape(_B, _K, _C)


# speedup vs baseline: 4.2767x; 1.0468x over previous
import functools

import jax
import jax.numpy as jnp
from jax import lax
from jax.experimental import pallas as pl
from jax.experimental.pallas import tpu as pltpu
from jax.experimental.pallas import tpu_sc as plsc

_K = 64
_B = 4
_S = 4096
_C = 1024
_MININT = -(2**31)

_NC = 2
_NS = 16
_NW = _NC * _NS
_CPW = _C // (_NW // _B)
_GRP = _CPW // 16


def _tc_threshold_body(x_ref, thr_ref, neq_ref):
    x = x_ref[0]
    b = lax.bitcast_convert_type(x, jnp.int32)
    s = jnp.where(b < 0, jnp.bitwise_xor(jnp.bitwise_not(b), _MININT), b)

    s4 = s.reshape(64, 64, 128)

    def _count_ge(c):
        m = (s4 >= c.reshape(1, 1, 128)).astype(jnp.int32)
        return jnp.sum(jnp.sum(m, axis=0), axis=0, keepdims=True)

    def cond(st):
        i, _, cur = st
        return jnp.logical_and(i < 32, jnp.logical_not(jnp.all(cur == _K)))

    def bit_step(st):
        i, prefix, cur = st
        bit = jnp.left_shift(jnp.int32(1), 31 - i)
        cand = jnp.bitwise_or(prefix, bit)
        cand_s = jnp.bitwise_xor(cand, _MININT)
        cnt = _count_ge(cand_s)
        acc = cnt >= _K
        return (i + 1, jnp.where(acc, cand, prefix), jnp.where(acc, cnt, cur))

    _, prefix, _ = lax.while_loop(
        cond, bit_step,
        (jnp.int32(0), jnp.zeros((1, 128), jnp.int32),
         jnp.full((1, 128), _S, jnp.int32)))
    prefix = jnp.where(prefix == jnp.int32(0x7FFFFFFF), prefix + 1, prefix)
    t_s = jnp.bitwise_xor(prefix, _MININT)
    m_gt = (s4 > t_s.reshape(1, 1, 128)).astype(jnp.int32)
    n_gt = jnp.sum(jnp.sum(m_gt, axis=0), axis=0, keepdims=True)
    b_thr = jnp.where(t_s < 0, jnp.bitwise_not(jnp.bitwise_xor(t_s, _MININT)), t_s)
    thr_ref[...] = lax.bitcast_convert_type(b_thr, jnp.float32)[None, None]
    neq_ref[...] = (_K - n_gt)[None, None]


_CHUNK = 512


def _sc_select_body(x_hbm, thr_hbm, neq_hbm, out_hbm, xbuf, obuf, tbuf, nbuf):
    wid = lax.axis_index("c") * _NS + lax.axis_index("s")
    b = wid // (_NW // _B)
    cbase = (wid % (_NW // _B)) * _CPW
    coff = b * _C + cbase
    lane = lax.iota(jnp.int32, 16)

    pltpu.sync_copy(thr_hbm.at[pl.ds(coff, _CPW)], tbuf)
    pltpu.sync_copy(neq_hbm.at[pl.ds(coff, _CPW)], nbuf)

    thrs = [tbuf[pl.ds(g * 16, 16)] for g in range(_GRP)]
    st0 = tuple(
        [(g * 16 + lane) * _K for g in range(_GRP)]
        + [nbuf[pl.ds(g * 16, 16)] for g in range(_GRP)])

    def chunk(ch, st):
        pltpu.sync_copy(
            x_hbm.at[b, pl.ds(ch * _CHUNK, _CHUNK), pl.ds(cbase, _CPW)], xbuf)

        def step(i, s2):
            addrs, rems = list(s2[:_GRP]), list(s2[_GRP:])
            for g in range(_GRP):
                v = xbuf[i, pl.ds(g * 16, 16)]
                gt = v > thrs[g]
                eq = v == thrs[g]
                sel = jnp.logical_or(gt, jnp.logical_and(eq, rems[g] > 0))
                plsc.store_scatter(obuf, [addrs[g]], v, mask=sel)
                rems[g] = rems[g] - eq.astype(jnp.int32)
                addrs[g] = addrs[g] + sel.astype(jnp.int32)
            return tuple(addrs + rems)

        return lax.fori_loop(0, _CHUNK, step, st)

    lax.fori_loop(0, _S // _CHUNK, chunk, st0)
    pltpu.sync_copy(obuf, out_hbm.at[pl.ds(wid * (_CPW * _K), _CPW * _K)])


@functools.cache
def _sc_select():
    return pl.kernel(
        _sc_select_body,
        out_type=jax.ShapeDtypeStruct((_NW * _CPW * _K,), jnp.float32),
        mesh=plsc.VectorSubcoreMesh(
            core_axis_name="c", subcore_axis_name="s",
            num_cores=_NC, num_subcores=_NS,
        ),
        compiler_params=pltpu.CompilerParams(needs_layout_passes=False),
        scratch_types=[
            pltpu.VMEM((_CHUNK, _CPW), jnp.float32),
            pltpu.VMEM((_CPW * _K,), jnp.float32),
            pltpu.VMEM((_CPW,), jnp.float32),
            pltpu.VMEM((_CPW,), jnp.int32),
        ],
    )


def kernel(x):
    thr4, neq4 = pl.pallas_call(
        _tc_threshold_body,
        grid=(_B, _C // 128),
        in_specs=[pl.BlockSpec((1, _S, 128), lambda b, cb: (b, 0, cb))],
        out_specs=[
            pl.BlockSpec((1, 1, 1, 128), lambda b, cb: (b, cb, 0, 0)),
            pl.BlockSpec((1, 1, 1, 128), lambda b, cb: (b, cb, 0, 0)),
        ],
        out_shape=[
            jax.ShapeDtypeStruct((_B, _C // 128, 1, 128), jnp.float32),
            jax.ShapeDtypeStruct((_B, _C // 128, 1, 128), jnp.int32),
        ],
    )(x)
    thr = thr4.reshape(_B * _C)
    neq = neq4.reshape(_B * _C)
    out_flat = _sc_select()(x, thr, neq)
    out = out_flat.reshape(_B, _NW // _B, _CPW, _K)
    return out.transpose(0, 3, 1, 2).reshape(_B, _K, _C)
